# Initial kernel scaffold; baseline (speedup 1.0000x reference)
import jax
import jax.numpy as jnp
from jax.experimental import pallas as pl


def kernel(x, W1, b1, W2, b2, W3, b3, meta_W, meta_b):
    raise NotImplementedError("write your pallas kernel here")


# speedup vs baseline: 1.3351x; 1.3351x over previous
import jax
import jax.numpy as jnp
from jax.experimental import pallas as pl
from jax.experimental.pallas import tpu as pltpu

_ROWS_PER_TILE = 256


def _layer_body(x_ref, w_ref, b_ref, s_ref, mw_ref, mb_ref, w_out_ref, h_ref):
    i = pl.program_id(0)
    w = w_ref[...]
    y = jax.lax.dot_general(
        x_ref[...], w, (((1,), (1,)), ((), ())),
        preferred_element_type=jnp.float32,
    )
    h = jnp.maximum(y + b_ref[...], 0.0)
    h_ref[...] = h
    w_out_ref[...] = w

    @pl.when(i == 0)
    def _fixup():
        cols_h = jax.lax.broadcasted_iota(jnp.int32, h.shape, 1)
        h0 = jnp.sum(jnp.where(cols_h == 0, h, 0.0))
        row0 = w_ref[0:1, :]
        cols_w = jax.lax.broadcasted_iota(jnp.int32, row0.shape, 1)
        w00 = jnp.sum(jnp.where(cols_w == 0, row0, 0.0))
        new00 = (s_ref[0, 0] * mw_ref[0, 0] + w00 * mw_ref[0, 1]
                 + h0 * mw_ref[0, 2] + mb_ref[0])
        w_out_ref[0:1, :] = jnp.where(cols_w == 0, new00, row0)


def _fused_layer(x, W, b2d, s, meta_W, meta_b):
    out_dim, in_dim = W.shape
    grid = (out_dim // _ROWS_PER_TILE,)
    w_new, h = pl.pallas_call(
        _layer_body,
        grid=grid,
        in_specs=[
            pl.BlockSpec((1, in_dim), lambda i: (0, 0)),
            pl.BlockSpec((_ROWS_PER_TILE, in_dim), lambda i: (i, 0)),
            pl.BlockSpec((1, _ROWS_PER_TILE), lambda i: (0, i)),
            pl.BlockSpec(memory_space=pltpu.SMEM),
            pl.BlockSpec(memory_space=pltpu.SMEM),
            pl.BlockSpec(memory_space=pltpu.SMEM),
        ],
        out_specs=[
            pl.BlockSpec((_ROWS_PER_TILE, in_dim), lambda i: (i, 0)),
            pl.BlockSpec((1, _ROWS_PER_TILE), lambda i: (0, i)),
        ],
        out_shape=[
            jax.ShapeDtypeStruct((out_dim, in_dim), jnp.float32),
            jax.ShapeDtypeStruct((1, out_dim), jnp.float32),
        ],
    )(x, W, b2d, s, meta_W, meta_b)
    return w_new, h


def kernel(x, W1, b1, W2, b2, W3, b3, meta_W, meta_b):
    b1r = b1.reshape(1, -1)
    b2r = b2.reshape(1, -1)
    b3r = b3.reshape(1, -1)
    W1n, h1 = _fused_layer(x, W1, b1r, x[0:1, 0:1], meta_W, meta_b)
    W2n, h2 = _fused_layer(h1, W2, b2r, h1[0:1, 0:1], meta_W, meta_b)
    W3n, out = _fused_layer(h2, W3, b3r, h2[0:1, 0:1], meta_W, meta_b)
    return (out, W1n, W2n, W3n)


# speedup vs baseline: 1.3744x; 1.0294x over previous
import jax
import jax.numpy as jnp
from jax.experimental import pallas as pl
from jax.experimental.pallas import tpu as pltpu

_ROWS_PER_TILE = 512


def _layer_body(x_ref, w_ref, b_ref, s_ref, mw_ref, mb_ref, w_out_ref, h_ref):
    i = pl.program_id(0)
    w = w_ref[...]
    y = jax.lax.dot_general(
        x_ref[...], w, (((1,), (1,)), ((), ())),
        preferred_element_type=jnp.float32,
    )
    h = jnp.maximum(y + b_ref[...], 0.0)
    h_ref[...] = h
    w_out_ref[...] = w

    @pl.when(i == 0)
    def _fixup():
        cols_h = jax.lax.broadcasted_iota(jnp.int32, h.shape, 1)
        h0 = jnp.sum(jnp.where(cols_h == 0, h, 0.0))
        row0 = w_ref[0:1, :]
        cols_w = jax.lax.broadcasted_iota(jnp.int32, row0.shape, 1)
        w00 = jnp.sum(jnp.where(cols_w == 0, row0, 0.0))
        new00 = (s_ref[0, 0] * mw_ref[0, 0] + w00 * mw_ref[0, 1]
                 + h0 * mw_ref[0, 2] + mb_ref[0])
        w_out_ref[0:1, :] = jnp.where(cols_w == 0, new00, row0)


def _fused_layer(x, W, b2d, s, meta_W, meta_b):
    out_dim, in_dim = W.shape
    grid = (out_dim // _ROWS_PER_TILE,)
    w_new, h = pl.pallas_call(
        _layer_body,
        grid=grid,
        in_specs=[
            pl.BlockSpec((1, in_dim), lambda i: (0, 0)),
            pl.BlockSpec((_ROWS_PER_TILE, in_dim), lambda i: (i, 0)),
            pl.BlockSpec((1, _ROWS_PER_TILE), lambda i: (0, i)),
            pl.BlockSpec(memory_space=pltpu.SMEM),
            pl.BlockSpec(memory_space=pltpu.SMEM),
            pl.BlockSpec(memory_space=pltpu.SMEM),
        ],
        out_specs=[
            pl.BlockSpec((_ROWS_PER_TILE, in_dim), lambda i: (i, 0)),
            pl.BlockSpec((1, _ROWS_PER_TILE), lambda i: (0, i)),
        ],
        out_shape=[
            jax.ShapeDtypeStruct((out_dim, in_dim), jnp.float32),
            jax.ShapeDtypeStruct((1, out_dim), jnp.float32),
        ],
    )(x, W, b2d, s, meta_W, meta_b)
    return w_new, h


def kernel(x, W1, b1, W2, b2, W3, b3, meta_W, meta_b):
    b1r = b1.reshape(1, -1)
    b2r = b2.reshape(1, -1)
    b3r = b3.reshape(1, -1)
    W1n, h1 = _fused_layer(x, W1, b1r, x[0:1, 0:1], meta_W, meta_b)
    W2n, h2 = _fused_layer(h1, W2, b2r, h1[0:1, 0:1], meta_W, meta_b)
    W3n, out = _fused_layer(h2, W3, b3r, h2[0:1, 0:1], meta_W, meta_b)
    return (out, W1n, W2n, W3n)


# speedup vs baseline: 1.5105x; 1.0991x over previous
import jax
import jax.numpy as jnp
from jax.experimental import pallas as pl
from jax.experimental.pallas import tpu as pltpu

_ROWS_PER_TILE = 1024


def _layer_body(x_ref, w_ref, b_ref, s_ref, mw_ref, mb_ref, w_out_ref, h_ref):
    i = pl.program_id(0)
    w = w_ref[...]
    y = jax.lax.dot_general(
        x_ref[...], w, (((1,), (1,)), ((), ())),
        preferred_element_type=jnp.float32,
    )
    h = jnp.maximum(y + b_ref[...], 0.0)
    h_ref[...] = h
    w_out_ref[...] = w

    @pl.when(i == 0)
    def _fixup():
        cols_h = jax.lax.broadcasted_iota(jnp.int32, h.shape, 1)
        h0 = jnp.sum(jnp.where(cols_h == 0, h, 0.0))
        row0 = w_ref[0:1, :]
        cols_w = jax.lax.broadcasted_iota(jnp.int32, row0.shape, 1)
        w00 = jnp.sum(jnp.where(cols_w == 0, row0, 0.0))
        new00 = (s_ref[0, 0] * mw_ref[0, 0] + w00 * mw_ref[0, 1]
                 + h0 * mw_ref[0, 2] + mb_ref[0])
        w_out_ref[0:1, :] = jnp.where(cols_w == 0, new00, row0)


def _fused_layer(x, W, b2d, s, meta_W, meta_b):
    out_dim, in_dim = W.shape
    grid = (out_dim // _ROWS_PER_TILE,)
    w_new, h = pl.pallas_call(
        _layer_body,
        grid=grid,
        in_specs=[
            pl.BlockSpec((1, in_dim), lambda i: (0, 0)),
            pl.BlockSpec((_ROWS_PER_TILE, in_dim), lambda i: (i, 0)),
            pl.BlockSpec((1, _ROWS_PER_TILE), lambda i: (0, i)),
            pl.BlockSpec(memory_space=pltpu.SMEM),
            pl.BlockSpec(memory_space=pltpu.SMEM),
            pl.BlockSpec(memory_space=pltpu.SMEM),
        ],
        out_specs=[
            pl.BlockSpec((_ROWS_PER_TILE, in_dim), lambda i: (i, 0)),
            pl.BlockSpec((1, _ROWS_PER_TILE), lambda i: (0, i)),
        ],
        out_shape=[
            jax.ShapeDtypeStruct((out_dim, in_dim), jnp.float32),
            jax.ShapeDtypeStruct((1, out_dim), jnp.float32),
        ],
    )(x, W, b2d, s, meta_W, meta_b)
    return w_new, h


def kernel(x, W1, b1, W2, b2, W3, b3, meta_W, meta_b):
    b1r = b1.reshape(1, -1)
    b2r = b2.reshape(1, -1)
    b3r = b3.reshape(1, -1)
    W1n, h1 = _fused_layer(x, W1, b1r, x[0:1, 0:1], meta_W, meta_b)
    W2n, h2 = _fused_layer(h1, W2, b2r, h1[0:1, 0:1], meta_W, meta_b)
    W3n, out = _fused_layer(h2, W3, b3r, h2[0:1, 0:1], meta_W, meta_b)
    return (out, W1n, W2n, W3n)


# speedup vs baseline: 1.7566x; 1.1629x over previous
import jax
import jax.numpy as jnp
from jax.experimental import pallas as pl
from jax.experimental.pallas import tpu as pltpu

_R = 512
_N = 2048
_BPW = _N // _R


def _body(x_ref, w1_ref, w2_ref, w3_ref, b1_ref, b2_ref, b3_ref,
          mw_ref, mb_ref, w1o_ref, w2o_ref, w3o_ref, out_ref,
          h1_ref, h2_ref):
    l = pl.program_id(0)
    t = pl.program_id(1)

    def compute(w_ref, b_ref, vec):
        wblk = w_ref[...]
        y = jax.lax.dot_general(
            vec, wblk, (((1,), (1,)), ((), ())),
            preferred_element_type=jnp.float32,
        )
        h = jnp.maximum(y + b_ref[0:1, pl.ds(t * _R, _R)], 0.0)
        return wblk, h

    def fixup(w_ref, w_out_ref, vec, h):
        cols_h = jax.lax.broadcasted_iota(jnp.int32, h.shape, 1)
        h0 = jnp.sum(jnp.where(cols_h == 0, h, 0.0))
        row0 = w_ref[0:1, :]
        cols_w = jax.lax.broadcasted_iota(jnp.int32, row0.shape, 1)
        w00 = jnp.sum(jnp.where(cols_w == 0, row0, 0.0))
        s = jnp.sum(jnp.where(cols_w == 0, vec, 0.0))
        new00 = (s * mw_ref[0, 0] + w00 * mw_ref[0, 1]
                 + h0 * mw_ref[0, 2] + mb_ref[0])
        w_out_ref[0:1, :] = jnp.where(cols_w == 0, new00, row0)

    @pl.when(l == 0)
    def _layer1():
        vec = x_ref[...]
        wblk, h = compute(w1_ref, b1_ref, vec)
        h1_ref[0:1, pl.ds(t * _R, _R)] = h
        w1o_ref[...] = wblk

        @pl.when(t == 0)
        def _():
            fixup(w1_ref, w1o_ref, vec, h)

    @pl.when(l == 1)
    def _layer2():
        vec = h1_ref[...]
        wblk, h = compute(w2_ref, b2_ref, vec)
        h2_ref[0:1, pl.ds(t * _R, _R)] = h
        w2o_ref[...] = wblk

        @pl.when(t == 0)
        def _():
            fixup(w2_ref, w2o_ref, vec, h)

    @pl.when(l == 2)
    def _layer3():
        vec = h2_ref[...]
        wblk, h = compute(w3_ref, b3_ref, vec)
        out_ref[...] = h
        w3o_ref[...] = wblk

        @pl.when(t == 0)
        def _():
            fixup(w3_ref, w3o_ref, vec, h)


def kernel(x, W1, b1, W2, b2, W3, b3, meta_W, meta_b):
    last = _BPW - 1
    w_spec = [
        pl.BlockSpec((_R, _N), lambda l, t: (jnp.where(l == 0, t, last), 0)),
        pl.BlockSpec((_R, _N), lambda l, t: (jnp.where(l == 0, 0, jnp.where(l == 1, t, last)), 0)),
        pl.BlockSpec((_R, _N), lambda l, t: (jnp.where(l == 2, t, 0), 0)),
    ]
    W1n, W2n, W3n, out = pl.pallas_call(
        _body,
        grid=(3, _BPW),
        in_specs=[
            pl.BlockSpec((1, _N), lambda l, t: (0, 0)),
            w_spec[0], w_spec[1], w_spec[2],
            pl.BlockSpec((1, _N), lambda l, t: (0, 0)),
            pl.BlockSpec((1, _N), lambda l, t: (0, 0)),
            pl.BlockSpec((1, _N), lambda l, t: (0, 0)),
            pl.BlockSpec(memory_space=pltpu.SMEM),
            pl.BlockSpec(memory_space=pltpu.SMEM),
        ],
        out_specs=[
            w_spec[0], w_spec[1], w_spec[2],
            pl.BlockSpec((1, _R), lambda l, t: (0, jnp.where(l == 2, t, 0))),
        ],
        out_shape=[
            jax.ShapeDtypeStruct((_N, _N), jnp.float32),
            jax.ShapeDtypeStruct((_N, _N), jnp.float32),
            jax.ShapeDtypeStruct((_N, _N), jnp.float32),
            jax.ShapeDtypeStruct((1, _N), jnp.float32),
        ],
        scratch_shapes=[
            pltpu.VMEM((1, _N), jnp.float32),
            pltpu.VMEM((1, _N), jnp.float32),
        ],
    )(x, W1, W2, W3, b1.reshape(1, -1), b2.reshape(1, -1),
      b3.reshape(1, -1), meta_W, meta_b)
    return (out, W1n, W2n, W3n)
